# baseline (speedup 1.0000x reference)
import functools

import jax
import jax.numpy as jnp
from jax import lax
from jax.experimental import pallas as pl
from jax.experimental.pallas import tpu as pltpu
from jax.experimental.pallas import tpu_sc as plsc

HIDDEN = 64
BATCH = 16384
NUM_ROWS = 1_000_000
TAIL_START = (NUM_ROWS // 128) * 128
W_TCS = 244
CHUNK_COLS = 256
NLANE = 16


@functools.cache
def _build_gather(batch: int, hidden: int):
    info = plsc.get_sparse_core_info()
    num_workers = info.num_cores * info.num_subcores
    cap = batch + NLANE
    mesh = plsc.VectorSubcoreMesh(core_axis_name="c", subcore_axis_name="s")

    @functools.partial(
        pl.kernel,
        mesh=mesh,
        out_type=jax.ShapeDtypeStruct((batch, hidden), jnp.float32),
        scratch_types=[
            pltpu.VMEM((batch,), jnp.int32),
            pltpu.VMEM((cap,), jnp.int32),
            pltpu.VMEM((cap,), jnp.int32),
            pltpu.VMEM((2, hidden, CHUNK_COLS), jnp.float32),
            pltpu.VMEM((hidden, 64), jnp.float32),
            pltpu.VMEM((NLANE, hidden), jnp.float32),
            pltpu.SemaphoreType.DMA,
            pltpu.SemaphoreType.DMA,
        ] + [pltpu.SemaphoreType.DMA] * NLANE,
        compiler_params=pltpu.CompilerParams(needs_layout_passes=False),
    )
    def gather_kernel(table_hbm, tail_hbm, idx_hbm, out_hbm, lab_all, labw_v,
                      pos_v, chunk_v, tail_v, row_v, *sems):
        csems = sems[:2]
        rsems = sems[2:]
        wid = lax.axis_index("s") * info.num_cores + lax.axis_index("c")
        zeros16 = jnp.zeros((NLANE,), jnp.int32)
        jvec = lax.iota(jnp.int32, NLANE)
        pltpu.sync_copy(idx_hbm, lab_all)
        pltpu.sync_copy(tail_hbm, tail_v)
        for l in range(NLANE):
            pltpu.async_copy(
                out_hbm.at[pl.ds(0, 1)], row_v.at[pl.ds(l, 1)], rsems[l]
            )

        widv = zeros16 + wid

        def build(i, off):
            lv = lab_all[pl.ds(i * NLANE, NLANE)]
            own = lax.min(
                lax.div(lax.shift_right_logical(lv, 7), W_TCS),
                num_workers - 1,
            )
            m = own == widv
            plsc.store_compressed(labw_v.at[pl.ds(off, NLANE)], lv, mask=m)
            plsc.store_compressed(
                pos_v.at[pl.ds(off, NLANE)], jvec + i * NLANE, mask=m
            )
            return off + plsc.all_reduce_population_count(m)[0]

        wl_len = lax.fori_loop(0, batch // NLANE, build, 0)
        labw_v[pl.ds(wl_len, NLANE)] = zeros16 - 1
        n_wvecs = lax.shift_right_logical(wl_len + NLANE - 1, 4)

        col0 = wid * (W_TCS * 128)
        is_last = wid == num_workers - 1
        n_chunks = jnp.where(is_last, (W_TCS * 128 + 512) // CHUNK_COLS,
                             W_TCS * 128 // CHUNK_COLS)

        def hit_rows(lv, pv, mi, s, get_col):
            @pl.when(s > 0)
            def _():
                for l in range(NLANE):
                    @pl.when(mi[l] == 1)
                    def _(l=l):
                        lab = lv[l]
                        pos = pv[l]
                        pltpu.make_async_copy(
                            out_hbm.at[pl.ds(0, 1)], row_v.at[pl.ds(l, 1)],
                            rsems[l],
                        ).wait()
                        get_col(lab, l)
                        pltpu.async_copy(
                            row_v.at[pl.ds(l, 1)], out_hbm.at[pl.ds(pos, 1)],
                            rsems[l],
                        )

        def issue_chunk(c, d):
            cc = lax.min(c, n_chunks - 1)
            start = pl.multiple_of(col0 + cc * CHUNK_COLS, 128)
            pltpu.async_copy(
                table_hbm.at[:, pl.ds(start, CHUNK_COLS)], chunk_v.at[d],
                csems[d],
            )

        issue_chunk(0, 0)
        issue_chunk(1, 1)

        def chunk_pair(g, carry):
            for d in range(2):
                c = 2 * g + d
                pltpu.make_async_copy(
                    table_hbm.at[:, pl.ds(0, CHUNK_COLS)], chunk_v.at[d],
                    csems[d],
                ).wait()
                c0 = col0 + c * CHUNK_COLS

                def scan(v, carry2, c0=c0, d=d):
                    lv = labw_v[pl.ds(v * NLANE, NLANE)]
                    pv = pos_v[pl.ds(v * NLANE, NLANE)]
                    mi = jnp.where((lv >= c0) & (lv < c0 + CHUNK_COLS), 1, 0)
                    s = jnp.sum(mi)

                    def get_col(lab, l, c0=c0, d=d):
                        cv = zeros16 + (lab - c0)
                        for jc in range(hidden // NLANE):
                            v16 = plsc.load_gather(
                                chunk_v.at[d], [jvec + NLANE * jc, cv]
                            )
                            row_v[l, pl.ds(NLANE * jc, NLANE)] = v16

                    hit_rows(lv, pv, mi, s, get_col)
                    return carry2

                lax.fori_loop(0, n_wvecs, scan, 0)
                issue_chunk(c + 2, d)
            return carry

        lax.fori_loop(0, (n_chunks + 1) // 2, chunk_pair, 0)

        def tail_scan(v, carry2):
            lv = labw_v[pl.ds(v * NLANE, NLANE)]
            pv = pos_v[pl.ds(v * NLANE, NLANE)]
            mi = jnp.where(lv >= TAIL_START, 1, 0)
            s = jnp.sum(mi)

            def get_col(lab, l):
                cv = zeros16 + (lab - TAIL_START)
                for jc in range(hidden // NLANE):
                    v16 = plsc.load_gather(tail_v, [jvec + NLANE * jc, cv])
                    row_v[l, pl.ds(NLANE * jc, NLANE)] = v16

            hit_rows(lv, pv, mi, s, get_col)
            return carry2

        lax.fori_loop(0, n_wvecs, tail_scan, 0)

        for d in range(2):
            pltpu.make_async_copy(
                table_hbm.at[:, pl.ds(0, CHUNK_COLS)], chunk_v.at[d], csems[d]
            ).wait()
        for l in range(NLANE):
            pltpu.make_async_copy(
                out_hbm.at[pl.ds(0, 1)], row_v.at[pl.ds(l, 1)], rsems[l]
            ).wait()

    return gather_kernel


def kernel(labels, train, embedding_table):
    del train
    gather = _build_gather(BATCH, HIDDEN)
    tail_t = embedding_table[TAIL_START:].T
    return gather(embedding_table.T, tail_t, labels.astype(jnp.int32))


# speedup vs baseline: 1.9431x; 1.9431x over previous
import functools

import jax
import jax.numpy as jnp
from jax import lax
from jax.experimental import pallas as pl
from jax.experimental.pallas import tpu as pltpu
from jax.experimental.pallas import tpu_sc as plsc

HIDDEN = 64
BATCH = 16384
NUM_ROWS = 1_000_000
TAIL_START = (NUM_ROWS // 128) * 128
TC_MAX = NUM_ROWS // 128 - 1
NBUF = 8


@functools.cache
def _build_gather(batch: int, hidden: int):
    info = plsc.get_sparse_core_info()
    num_workers = info.num_cores * info.num_subcores
    b_per_w = batch // num_workers
    lab_pad = b_per_w + 32
    mesh = plsc.VectorSubcoreMesh(core_axis_name="c", subcore_axis_name="s")

    @functools.partial(
        pl.kernel,
        mesh=mesh,
        out_type=jax.ShapeDtypeStruct((hidden, batch), jnp.float32),
        scratch_types=[
            pltpu.VMEM((lab_pad,), jnp.int32),
            pltpu.VMEM((hidden, 64), jnp.float32),
            pltpu.VMEM((NBUF, hidden, 128), jnp.float32),
            pltpu.VMEM((hidden, b_per_w), jnp.float32),
            pltpu.SemaphoreType.DMA,
            pltpu.SemaphoreType.DMA,
            pltpu.SemaphoreType.DMA,
            pltpu.SemaphoreType.DMA,
            pltpu.SemaphoreType.DMA,
            pltpu.SemaphoreType.DMA,
            pltpu.SemaphoreType.DMA,
            pltpu.SemaphoreType.DMA,
        ],
        compiler_params=pltpu.CompilerParams(needs_layout_passes=False),
    )
    def gather_kernel(table_hbm, tail_hbm, idx_hbm, out_hbm, lab_v, tail_v,
                      slab_v, outb_v, *sems):
        wid = lax.axis_index("s") * info.num_cores + lax.axis_index("c")
        base = wid * b_per_w
        zeros16 = jnp.zeros((16,), jnp.int32)
        for i in range((lab_pad - b_per_w) // 16):
            lab_v[pl.ds(b_per_w + 16 * i, 16)] = zeros16
        pltpu.sync_copy(idx_hbm.at[pl.ds(base, b_per_w)],
                        lab_v.at[pl.ds(0, b_per_w)])
        pltpu.sync_copy(tail_hbm, tail_v)
        jvec = lax.iota(jnp.int32, 16)

        def issue(lab, slot):
            tc = lax.min(lax.shift_right_logical(lab, 7), TC_MAX)
            start = pl.multiple_of(tc * 128, 128)
            pltpu.async_copy(
                table_hbm.at[:, pl.ds(start, 128)], slab_v.at[slot],
                sems[slot],
            )

        def wait(slot):
            pltpu.make_async_copy(
                table_hbm.at[:, pl.ds(0, 128)], slab_v.at[slot], sems[slot]
            ).wait()

        def extract(lab, a, slot):
            avec = zeros16 + a
            is_tail = lab >= TAIL_START

            @pl.when(is_tail)
            def _():
                ctv = zeros16 + (lab - TAIL_START)
                for jc in range(hidden // 16):
                    v = plsc.load_gather(tail_v, [jvec + 16 * jc, ctv])
                    plsc.store_scatter(outb_v, [jvec + 16 * jc, avec], v)

            @pl.when(jnp.logical_not(is_tail))
            def _():
                cvec = zeros16 + (lab & 127)
                for jc in range(hidden // 16):
                    v = plsc.load_gather(
                        slab_v.at[slot], [jvec + 16 * jc, cvec]
                    )
                    plsc.store_scatter(outb_v, [jvec + 16 * jc, avec], v)

        vec0 = lab_v[pl.ds(0, 16)]
        for d in range(NBUF):
            issue(vec0[d], d)

        def group(g, carry):
            vecg = lab_v[pl.ds(g * NBUF, 16)]
            for d in range(NBUF):
                wait(d)
                extract(vecg[d], g * NBUF + d, d)
                issue(vecg[d + NBUF], d)
            return carry

        lax.fori_loop(0, b_per_w // NBUF, group, 0)
        for d in range(NBUF):
            wait(d)
        pltpu.sync_copy(outb_v, out_hbm.at[:, pl.ds(base, b_per_w)])

    return gather_kernel


def kernel(labels, train, embedding_table):
    del train
    gather = _build_gather(BATCH, HIDDEN)
    tail_t = embedding_table[TAIL_START:].T
    out_t = gather(embedding_table.T, tail_t, labels.astype(jnp.int32))
    return out_t.T
